# baseline (speedup 1.0000x reference)
import jax
import jax.numpy as jnp
from jax.experimental import pallas as pl
from jax.experimental.pallas import tpu as pltpu

_TB = 512
_VMEM_LIMIT = 64 * 1024 * 1024


def _round_up(x, m):
    return -(-x // m) * m


def _net_kernel(x_ref, t1_ref, t1e_ref, b1_ref, t2_ref, b2_ref, wb_ref,
                bl1_ref, wl2_ref, bl2_ref, wl3_ref, bl3_ref, o_ref, p_ref):
    tb = o_ref.shape[0]
    f32 = jnp.float32
    bf16 = jnp.bfloat16

    b1 = b1_ref[...]
    for u in range(0, 12, 2):
        base = max(28 * (2 * u - 1), 0)
        xu = x_ref[:, base:base + 224].astype(bf16)
        t1 = t1e_ref[...] if u == 0 else t1_ref[...]
        c1 = jnp.dot(xu, t1, preferred_element_type=f32) + b1
        for w in range(2):
            cw = c1[:, 1024 * w:1024 * (w + 1)]
            pu = (jnp.maximum(cw[:, 0:256], 0.0)
                  + jnp.maximum(cw[:, 256:512], 0.0)
                  + jnp.maximum(cw[:, 512:768], 0.0)
                  + jnp.maximum(cw[:, 768:1024], 0.0))
            for ky in range(3):
                r = u + w - 2 * ky
                if 0 <= r < 8:
                    p_ref[ky, :, r, :] = pu

    z = None
    for ky in range(3):
        slab = p_ref[ky].reshape(tb * 8, 256).astype(bf16)
        zk = jnp.dot(slab, t2_ref[ky], preferred_element_type=f32)
        z = zk if z is None else z + zk
    z = jnp.maximum(z + b2_ref[...], 0.0)
    z3 = z.astype(bf16).reshape(tb, 8, 256)

    h = bl1_ref[...]
    for i in range(8):
        h = h + jnp.dot(z3[:, i, :], wb_ref[i], preferred_element_type=f32)
    h = jnp.maximum(h, 0.0)

    h2 = jnp.dot(h.astype(bf16), wl2_ref[...], preferred_element_type=f32)
    h2 = jnp.maximum(h2 + bl2_ref[...], 0.0)

    o_ref[...] = (jnp.dot(h2.astype(bf16), wl3_ref[...],
                          preferred_element_type=f32) + bl3_ref[...])


def _const_index_map(nd):
    return lambda i, _nd=nd: (0,) * _nd


def _prep_weights(w1p, b1p, w2p, b2p, wl1p):
    f32 = jnp.float32
    bf16 = jnp.bfloat16
    def band(shift):
        rho = jnp.arange(8)[None, None, :, None]
        ky = jnp.arange(3)[None, :, None, None]
        dy = jnp.arange(2)[:, None, None, None]
        w_ = jnp.arange(2)[None, None, None, :]
        ey = (rho == 2 * (w_ + ky) + dy - shift).astype(f32)
        s = jnp.arange(28)[None, None, :, None]
        v = jnp.arange(16)[None, None, None, :]
        kx = jnp.arange(3)[None, :, None, None]
        dx = jnp.arange(2)[:, None, None, None]
        ex = ((s == 2 * (v + kx) + dx - 1) & (v < 12)).astype(f32)
        w1r = w1p.reshape(3, 3, 16)
        t = jnp.einsum('darw,ebsv,abc->rswdevc', ey, ex, w1r)
        return t.reshape(224, 2048).astype(bf16)
    t1 = band(0)
    t1e = band(1)
    b1t = jnp.tile(b1p.reshape(1, 16), (1, 128))
    e2 = (jnp.arange(16)[None, :, None]
          == jnp.arange(8)[None, None, :] + 2 * jnp.arange(3)[:, None, None])
    e2 = e2.astype(f32)
    w2r = w2p.reshape(3, 3, 16, 32)
    t2 = 0.25 * jnp.einsum('bvj,abcd->avcjd', e2, w2r)
    t2 = t2.reshape(3, 256, 256)
    b2t = jnp.tile(b2p.reshape(1, 32), (1, 8))
    ii = jnp.arange(8)
    pos = ((ii[:, None] // 2) * 4 + (ii[None, :] // 2)).reshape(64)
    wb = (wl1p[pos] * 0.25).reshape(8, 256, 256)
    return (t1, t1e, b1t.astype(f32), t2.astype(bf16),
            b2t.astype(f32), wb.astype(bf16))


def kernel(x_nchw, w1p, b1p, w2p, b2p, wl1p, bl1p, wl2p, bl2p, wl3p, bl3p):
    bsz = x_nchw.shape[0]
    tb = min(_TB, _round_up(bsz, 8))
    bp = _round_up(bsz, tb)
    nb = bp // tb

    t1, t1e, b1t, t2, b2t, wb = _prep_weights(w1p, b1p, w2p, b2p, wl1p)
    xflat = x_nchw.reshape(bsz, 784)
    if bp != bsz:
        xflat = jnp.pad(xflat, ((0, bp - bsz), (0, 0)))

    weights = (t1, t1e, b1t, t2, b2t, wb, bl1p.astype(jnp.float32),
               wl2p.astype(jnp.bfloat16), bl2p.astype(jnp.float32),
               wl3p[:, :10].astype(jnp.bfloat16),
               bl3p[:, :10].astype(jnp.float32))

    logits = pl.pallas_call(
        _net_kernel,
        out_shape=jax.ShapeDtypeStruct((bp, 10), jnp.float32),
        grid=(nb,),
        in_specs=[pl.BlockSpec((tb, 784), lambda i: (i, 0))]
                 + [pl.BlockSpec(w.shape, _const_index_map(w.ndim))
                    for w in weights],
        out_specs=pl.BlockSpec((tb, 10), lambda i: (i, 0)),
        scratch_shapes=[pltpu.VMEM((3, tb, 8, 256), jnp.float32)],
        compiler_params=pltpu.CompilerParams(
            dimension_semantics=("parallel",),
            vmem_limit_bytes=_VMEM_LIMIT),
    )(xflat, *weights)
    return logits[:bsz]


# speedup vs baseline: 1.0351x; 1.0351x over previous
import jax
import jax.numpy as jnp
from jax.experimental import pallas as pl
from jax.experimental.pallas import tpu as pltpu

_TB = 512
_VMEM_LIMIT = 64 * 1024 * 1024


def _round_up(x, m):
    return -(-x // m) * m


def _net_kernel(x_ref, t1_ref, t1e_ref, b1_ref, t2_ref, b2_ref, wb_ref,
                bl1_ref, wl2_ref, bl2_ref, wl3_ref, bl3_ref, o_ref, p_ref):
    tb = o_ref.shape[0]
    f32 = jnp.float32
    bf16 = jnp.bfloat16

    b1 = b1_ref[...]
    for u in range(0, 12, 2):
        base = max(28 * (2 * u - 1), 0)
        xu = x_ref[:, base:base + 224].astype(bf16)
        t1 = t1e_ref[...] if u == 0 else t1_ref[...]
        c1 = jnp.dot(xu, t1, preferred_element_type=f32) + b1
        for w in range(2):
            cw = c1[:, 768 * w:768 * (w + 1)]
            pu = (jnp.maximum(cw[:, 0:192], 0.0)
                  + jnp.maximum(cw[:, 192:384], 0.0)
                  + jnp.maximum(cw[:, 384:576], 0.0)
                  + jnp.maximum(cw[:, 576:768], 0.0))
            for ky in range(3):
                r = u + w - 2 * ky
                if 0 <= r < 8:
                    p_ref[ky, :, r, :] = pu

    z = None
    for ky in range(3):
        slab = p_ref[ky].reshape(tb * 8, 192).astype(bf16)
        zk = jnp.dot(slab, t2_ref[ky], preferred_element_type=f32)
        z = zk if z is None else z + zk
    z = jnp.maximum(z + b2_ref[...], 0.0)
    z3 = z.astype(bf16).reshape(tb, 8, 256)

    h = bl1_ref[...]
    for i in range(8):
        h = h + jnp.dot(z3[:, i, :], wb_ref[i], preferred_element_type=f32)
    h = jnp.maximum(h, 0.0)

    h2 = jnp.dot(h.astype(bf16), wl2_ref[...], preferred_element_type=f32)
    h2 = jnp.maximum(h2 + bl2_ref[...], 0.0)

    o_ref[...] = (jnp.dot(h2.astype(bf16), wl3_ref[...],
                          preferred_element_type=f32) + bl3_ref[...])


def _const_index_map(nd):
    return lambda i, _nd=nd: (0,) * _nd


def _prep_weights(w1p, b1p, w2p, b2p, wl1p):
    f32 = jnp.float32
    bf16 = jnp.bfloat16
    def band(shift):
        rho = jnp.arange(8)[None, None, :, None]
        ky = jnp.arange(3)[None, :, None, None]
        dy = jnp.arange(2)[:, None, None, None]
        w_ = jnp.arange(2)[None, None, None, :]
        ey = (rho == 2 * (w_ + ky) + dy - shift).astype(f32)
        s = jnp.arange(28)[None, None, :, None]
        v = jnp.arange(12)[None, None, None, :]
        kx = jnp.arange(3)[None, :, None, None]
        dx = jnp.arange(2)[:, None, None, None]
        ex = (s == 2 * (v + kx) + dx - 1).astype(f32)
        w1r = w1p.reshape(3, 3, 16)
        t = jnp.einsum('darw,ebsv,abc->rswdevc', ey, ex, w1r)
        return t.reshape(224, 1536).astype(bf16)
    t1 = band(0)
    t1e = band(1)
    b1t = jnp.tile(b1p.reshape(1, 16), (1, 96))
    e2 = (jnp.arange(12)[None, :, None]
          == jnp.arange(8)[None, None, :] + 2 * jnp.arange(3)[:, None, None])
    e2 = e2.astype(f32)
    w2r = w2p.reshape(3, 3, 16, 32)
    t2 = 0.25 * jnp.einsum('bvj,abcd->avcjd', e2, w2r)
    t2 = t2.reshape(3, 192, 256)
    b2t = jnp.tile(b2p.reshape(1, 32), (1, 8))
    ii = jnp.arange(8)
    pos = ((ii[:, None] // 2) * 4 + (ii[None, :] // 2)).reshape(64)
    wb = (wl1p[pos] * 0.25).reshape(8, 256, 256)
    return (t1, t1e, b1t.astype(f32), t2.astype(bf16),
            b2t.astype(f32), wb.astype(bf16))


def kernel(x_nchw, w1p, b1p, w2p, b2p, wl1p, bl1p, wl2p, bl2p, wl3p, bl3p):
    bsz = x_nchw.shape[0]
    tb = min(_TB, _round_up(bsz, 8))
    bp = _round_up(bsz, tb)
    nb = bp // tb

    t1, t1e, b1t, t2, b2t, wb = _prep_weights(w1p, b1p, w2p, b2p, wl1p)
    xflat = x_nchw.reshape(bsz, 784)
    if bp != bsz:
        xflat = jnp.pad(xflat, ((0, bp - bsz), (0, 0)))

    weights = (t1, t1e, b1t, t2, b2t, wb, bl1p.astype(jnp.float32),
               wl2p.astype(jnp.bfloat16), bl2p.astype(jnp.float32),
               wl3p[:, :10].astype(jnp.bfloat16),
               bl3p[:, :10].astype(jnp.float32))

    logits = pl.pallas_call(
        _net_kernel,
        out_shape=jax.ShapeDtypeStruct((bp, 10), jnp.float32),
        grid=(nb,),
        in_specs=[pl.BlockSpec((tb, 784), lambda i: (i, 0))]
                 + [pl.BlockSpec(w.shape, _const_index_map(w.ndim))
                    for w in weights],
        out_specs=pl.BlockSpec((tb, 10), lambda i: (i, 0)),
        scratch_shapes=[pltpu.VMEM((3, tb, 8, 192), jnp.float32)],
        compiler_params=pltpu.CompilerParams(
            dimension_semantics=("parallel",),
            vmem_limit_bytes=_VMEM_LIMIT),
    )(xflat, *weights)
    return logits[:bsz]


# speedup vs baseline: 1.0680x; 1.0318x over previous
import jax
import jax.numpy as jnp
from jax.experimental import pallas as pl
from jax.experimental.pallas import tpu as pltpu

_TB = 512
_VMEM_LIMIT = 64 * 1024 * 1024


def _round_up(x, m):
    return -(-x // m) * m


def _net_kernel(x_ref, t1_ref, b1_ref, t2_ref, b2_ref, wb_ref,
                bl1_ref, wl2_ref, bl2_ref, wl3_ref, bl3_ref, o_ref, p_ref):
    tb = o_ref.shape[0]
    f32 = jnp.float32
    bf16 = jnp.bfloat16

    t1 = t1_ref[...]
    b1 = b1_ref[...]
    for u in range(0, 12, 2):
        if u == 0:
            xu = x_ref[:, 0:196].astype(bf16)
            c1 = jnp.dot(xu, t1[28:224], preferred_element_type=f32) + b1
        else:
            base = 28 * (2 * u - 1)
            xu = x_ref[:, base:base + 224].astype(bf16)
            c1 = jnp.dot(xu, t1, preferred_element_type=f32) + b1
        for w in range(2):
            cw = c1[:, 768 * w:768 * (w + 1)]
            pu = (jnp.maximum(cw[:, 0:192], 0.0)
                  + jnp.maximum(cw[:, 192:384], 0.0)
                  + jnp.maximum(cw[:, 384:576], 0.0)
                  + jnp.maximum(cw[:, 576:768], 0.0))
            for ky in range(3):
                r = u + w - 2 * ky
                if 0 <= r < 8:
                    p_ref[ky, :, r, :] = pu

    z = None
    for ky in range(3):
        slab = p_ref[ky].reshape(tb * 8, 192).astype(bf16)
        zk = jnp.dot(slab, t2_ref[ky], preferred_element_type=f32)
        z = zk if z is None else z + zk
    z = jnp.maximum(z + b2_ref[...], 0.0)
    z3 = z.astype(bf16).reshape(tb, 8, 256)

    h = bl1_ref[...]
    for i in range(8):
        h = h + jnp.dot(z3[:, i, :], wb_ref[i], preferred_element_type=f32)
    h = jnp.maximum(h, 0.0)

    h2 = jnp.dot(h.astype(bf16), wl2_ref[...], preferred_element_type=f32)
    h2 = jnp.maximum(h2 + bl2_ref[...], 0.0)

    o_ref[...] = (jnp.dot(h2.astype(bf16), wl3_ref[...],
                          preferred_element_type=f32) + bl3_ref[...])


def _const_index_map(nd):
    return lambda i, _nd=nd: (0,) * _nd


def _prep_weights(w1p, b1p, w2p, b2p, wl1p):
    f32 = jnp.float32
    bf16 = jnp.bfloat16
    rho = jnp.arange(8)[None, None, :, None]
    ky = jnp.arange(3)[None, :, None, None]
    dy = jnp.arange(2)[:, None, None, None]
    w_ = jnp.arange(2)[None, None, None, :]
    ey = (rho == 2 * (w_ + ky) + dy).astype(f32)
    s = jnp.arange(28)[None, None, :, None]
    v = jnp.arange(12)[None, None, None, :]
    kx = jnp.arange(3)[None, :, None, None]
    dx = jnp.arange(2)[:, None, None, None]
    ex = (s == 2 * (v + kx) + dx - 1).astype(f32)
    w1r = w1p.reshape(3, 3, 16)
    t1 = jnp.einsum('darw,ebsv,abc->rswdevc', ey, ex, w1r)
    t1 = t1.reshape(224, 1536).astype(bf16)
    b1t = jnp.tile(b1p.reshape(1, 16), (1, 96))
    e2 = (jnp.arange(12)[None, :, None]
          == jnp.arange(8)[None, None, :] + 2 * jnp.arange(3)[:, None, None])
    e2 = e2.astype(f32)
    w2r = w2p.reshape(3, 3, 16, 32)
    t2 = 0.25 * jnp.einsum('bvj,abcd->avcjd', e2, w2r)
    t2 = t2.reshape(3, 192, 256)
    b2t = jnp.tile(b2p.reshape(1, 32), (1, 8))
    ii = jnp.arange(8)
    pos = ((ii[:, None] // 2) * 4 + (ii[None, :] // 2)).reshape(64)
    wb = (wl1p[pos] * 0.25).reshape(8, 256, 256)
    return (t1, b1t.astype(f32), t2.astype(bf16),
            b2t.astype(f32), wb.astype(bf16))


def kernel(x_nchw, w1p, b1p, w2p, b2p, wl1p, bl1p, wl2p, bl2p, wl3p, bl3p):
    bsz = x_nchw.shape[0]
    tb = min(_TB, _round_up(bsz, 8))
    bp = _round_up(bsz, tb)
    nb = bp // tb

    t1, b1t, t2, b2t, wb = _prep_weights(w1p, b1p, w2p, b2p, wl1p)
    xflat = x_nchw.reshape(bsz, 784)
    if bp != bsz:
        xflat = jnp.pad(xflat, ((0, bp - bsz), (0, 0)))

    weights = (t1, b1t, t2, b2t, wb, bl1p.astype(jnp.float32),
               wl2p.astype(jnp.bfloat16), bl2p.astype(jnp.float32),
               wl3p[:, :10].astype(jnp.bfloat16),
               bl3p[:, :10].astype(jnp.float32))

    logits = pl.pallas_call(
        _net_kernel,
        out_shape=jax.ShapeDtypeStruct((bp, 10), jnp.float32),
        grid=(nb,),
        in_specs=[pl.BlockSpec((tb, 784), lambda i: (i, 0))]
                 + [pl.BlockSpec(w.shape, _const_index_map(w.ndim))
                    for w in weights],
        out_specs=pl.BlockSpec((tb, 10), lambda i: (i, 0)),
        scratch_shapes=[pltpu.VMEM((3, tb, 8, 192), jnp.float32)],
        compiler_params=pltpu.CompilerParams(
            dimension_semantics=("parallel",),
            vmem_limit_bytes=_VMEM_LIMIT),
    )(xflat, *weights)
    return logits[:bsz]
